# baseline (speedup 1.0000x reference)
import functools

import jax
import jax.numpy as jnp
from jax import lax
from jax.experimental import pallas as pl
from jax.experimental.pallas import tpu as pltpu
from jax.experimental.pallas import tpu_sc as plsc

OUT = 64
B = 16384
CONST_VAL = 0.1

NC = 2
NS = 16
L = 16
NW = NC * NS
BPW = B // NW


_mesh = plsc.VectorSubcoreMesh(core_axis_name="c", subcore_axis_name="s")


@functools.partial(
    pl.kernel,
    mesh=_mesh,
    compiler_params=pltpu.CompilerParams(use_tc_tiling_on_sc=False),
    out_type=(
        jax.ShapeDtypeStruct((B, OUT), jnp.float32),
        jax.ShapeDtypeStruct((B, OUT), jnp.float32),
        jax.ShapeDtypeStruct((B * OUT,), jnp.float32),
    ),
    scratch_types=[
        pltpu.VMEM((BPW,), jnp.int32),
        pltpu.VMEM((BPW, OUT), jnp.float32),
        pltpu.VMEM((BPW,), jnp.int32),
        pltpu.VMEM((BPW, OUT), jnp.float32),
        pltpu.VMEM((BPW * OUT,), jnp.float32),
        pltpu.SemaphoreType.DMA,
        pltpu.SemaphoreType.DMA,
    ],
)
def _sc_encode(uid_hbm, iid_hbm, wu_hbm, wi_hbm,
               ou_hbm, oi_hbm, oc_hbm,
               uidx_v, urows_v, iidx_v, irows_v, ctx_v,
               sem_u, sem_i):
    wid = lax.axis_index("s") * NC + lax.axis_index("c")
    base = wid * BPW

    pltpu.sync_copy(uid_hbm.at[pl.ds(base, BPW)], uidx_v)
    pltpu.sync_copy(iid_hbm.at[pl.ds(base, BPW)], iidx_v)

    cu = pltpu.async_copy(wu_hbm.at[uidx_v], urows_v, sem_u)
    ci = pltpu.async_copy(wi_hbm.at[iidx_v], irows_v, sem_i)

    cvec = jnp.full((L,), CONST_VAL, jnp.float32)

    def fill(i, _):
        ctx_v[pl.ds(i * L, L)] = cvec
        return ()

    lax.fori_loop(0, (BPW * OUT) // L, fill, (), unroll=8)
    pltpu.sync_copy(ctx_v, oc_hbm.at[pl.ds(base * OUT, BPW * OUT)])

    cu.wait()
    pltpu.sync_copy(urows_v, ou_hbm.at[pl.ds(base, BPW)])
    ci.wait()
    pltpu.sync_copy(irows_v, oi_hbm.at[pl.ds(base, BPW)])


@jax.jit
def kernel(user_ids, item_ids, context_ids, W_user, W_item):
    del context_ids
    user_emb, item_emb, ctx_flat = _sc_encode(
        user_ids.astype(jnp.int32), item_ids.astype(jnp.int32),
        W_user, W_item)
    return user_emb, item_emb, ctx_flat.reshape(B, OUT)


# speedup vs baseline: 1.8294x; 1.8294x over previous
import functools

import jax
import jax.numpy as jnp
from jax import lax
from jax.experimental import pallas as pl
from jax.experimental.pallas import tpu as pltpu
from jax.experimental.pallas import tpu_sc as plsc

OUT = 64
B = 16384
CONST_VAL = 0.1
VU = 1000000
VI = 100000

NC = 2
NS = 16
L = 16
NW = NC * NS

W = 512
NFULL_U = VU // W
NFULL_I = VI // W
ALIGN_I = 99840
PAD_U = 999936
PAD_I = 99968
CBUF = 2048

_mesh = plsc.VectorSubcoreMesh(core_axis_name="c", subcore_axis_name="s")


@functools.partial(
    pl.kernel,
    mesh=_mesh,
    compiler_params=pltpu.CompilerParams(
        use_tc_tiling_on_sc=True, needs_layout_passes=False),
    out_type=(
        jax.ShapeDtypeStruct((B * OUT,), jnp.float32),
        jax.ShapeDtypeStruct((B * OUT,), jnp.float32),
        jax.ShapeDtypeStruct((B * OUT,), jnp.float32),
    ),
    scratch_types=[
        pltpu.VMEM((B,), jnp.int32),
        pltpu.VMEM((B + L,), jnp.int32),
        pltpu.VMEM((B + L,), jnp.int32),
        pltpu.VMEM((B + L,), jnp.int32),
        pltpu.VMEM((B + L,), jnp.int32),
        pltpu.VMEM((OUT, W), jnp.float32),
        pltpu.VMEM((L * OUT,), jnp.float32),
        pltpu.VMEM((L * OUT,), jnp.int32),
        pltpu.VMEM((CBUF,), jnp.float32),
        pltpu.SemaphoreType.DMA,
    ],
)
def _sc_encode(uid_hbm, iid_hbm, wut_hbm, wit_hbm, upad_hbm, ipad_hbm,
               ou_hbm, oi_hbm, oc_hbm,
               ids_v, mi_v, mb_v, wc_v, wb_v, buf_v, ring_v, drain_v, cb_v,
               sem):
    wid = lax.axis_index("s") * NC + lax.axis_index("c")
    lanes = lax.iota(jnp.int32, L)

    cvec = jnp.full((L,), CONST_VAL, jnp.float32)

    def cfill(i, _):
        cb_v[pl.ds(i * L, L)] = cvec
        return ()

    lax.fori_loop(0, CBUF // L, cfill, (), unroll=8)
    base_c = wid * (B * OUT // NW)
    for j in range(B * OUT // NW // CBUF):
        pltpu.sync_copy(cb_v, oc_hbm.at[pl.ds(base_c + j * CBUF, CBUF)])

    def run_table(idx_hbm, wt_hbm, pad_hbm, out_hbm, nfull, align_base,
                  pad_base):
        pltpu.sync_copy(idx_hbm, ids_v)

        def pf(i, cnt):
            idv = ids_v[pl.ds(i * L, L)]
            bv = lanes + i * L
            m = (idv // W) % NW == wid
            pos = plsc.cumsum(m.astype(jnp.int32))
            dst = jnp.where(m, cnt + pos - 1, B)
            plsc.store_scatter(mi_v, [dst], idv)
            plsc.store_scatter(mb_v, [dst], bv)
            return cnt + jnp.max(pos)

        cnt = lax.fori_loop(0, B // L, pf, jnp.int32(0))
        nch = (cnt + L - 1) // L

        def process_window(base, match_w, read_w):
            if read_w == W:
                pltpu.sync_copy(wt_hbm.at[:, pl.ds(base, W)], buf_v)
            elif read_w:
                pltpu.sync_copy(wt_hbm.at[:, pl.ds(base, read_w)],
                                buf_v.at[:, pl.ds(0, read_w)])

            def wf(j, k):
                idv = mi_v[pl.ds(j * L, L)]
                bv = mb_v[pl.ds(j * L, L)]
                valid = (j * L + lanes) < cnt
                m = valid & (idv >= base) & (idv < base + match_w)
                pos = plsc.cumsum(m.astype(jnp.int32))
                dst = jnp.where(m, k + pos - 1, B)
                plsc.store_scatter(wc_v, [dst], idv - base)
                plsc.store_scatter(wb_v, [dst], bv)
                return k + jnp.max(pos)

            kw = lax.fori_loop(0, nch, wf, jnp.int32(0))

            def grp(t16, _):
                colv0 = wc_v[pl.ds(t16 * L, L)]
                bv0 = wb_v[pl.ds(t16 * L, L)]
                rem = kw - t16 * L
                colv = jnp.where(lanes < rem, colv0, 0)
                for l in range(L):
                    @pl.when(l < rem)
                    def _issue():
                        col = jnp.max(jnp.where(lanes == l, colv, 0))
                        b = jnp.max(jnp.where(lanes == l, bv0, 0))
                        for c0 in range(0, OUT, L):
                            ring_v[pl.ds(l * OUT + c0, L)] = plsc.load_gather(
                                buf_v, [lanes + c0, jnp.broadcast_to(col, (L,))])
                        pltpu.async_copy(ring_v.at[pl.ds(l * OUT, OUT)],
                                         out_hbm.at[pl.ds(b * OUT, OUT)], sem)

                    @pl.when(l >= rem)
                    def _dummy():
                        pltpu.async_copy(idx_hbm.at[pl.ds(l * OUT, OUT)],
                                         drain_v.at[pl.ds(l * OUT, OUT)], sem)
                pltpu.make_async_copy(idx_hbm.at[pl.ds(0, L * OUT)],
                                      drain_v, sem).wait()
                return ()

            lax.fori_loop(0, (kw + L - 1) // L, grp, ())

        @pl.when(wid == (pad_base // W) % NW)
        def _pad_station():
            pltpu.sync_copy(pad_hbm, buf_v.at[:, pl.ds(0, 128)])
            process_window(jnp.int32(pad_base), W - (pad_base % W), 0)

        if align_base is not None:
            @pl.when(wid == (align_base // W) % NW)
            def _align_station():
                process_window(jnp.int32(align_base), 128, 128)

        nfull_me = (nfull + (NW - 1) - wid) // NW

        def wloop(lw, _):
            g = wid + lw * NW
            process_window(g * W, W, W)
            return ()

        lax.fori_loop(0, nfull_me, wloop, ())

    run_table(uid_hbm, wut_hbm, upad_hbm, ou_hbm, NFULL_U, None, PAD_U)
    run_table(iid_hbm, wit_hbm, ipad_hbm, oi_hbm, NFULL_I, ALIGN_I, PAD_I)


@jax.jit
def kernel(user_ids, item_ids, context_ids, W_user, W_item):
    del context_ids
    upad = jnp.pad(W_user[PAD_U:].T, ((0, 0), (0, 128 - (VU - PAD_U))))
    ipad = jnp.pad(W_item[PAD_I:].T, ((0, 0), (0, 128 - (VI - PAD_I))))
    u_flat, i_flat, c_flat = _sc_encode(
        user_ids.astype(jnp.int32), item_ids.astype(jnp.int32),
        jnp.transpose(W_user), jnp.transpose(W_item), upad, ipad)
    user_emb = u_flat.reshape(B, OUT)
    item_emb = i_flat.reshape(B, OUT)
    ctx_emb = c_flat.reshape(OUT, B).T
    return user_emb, item_emb, ctx_emb


# speedup vs baseline: 2.1960x; 1.2004x over previous
import functools

import jax
import jax.numpy as jnp
from jax import lax
from jax.experimental import pallas as pl
from jax.experimental.pallas import tpu as pltpu
from jax.experimental.pallas import tpu_sc as plsc

OUT = 64
B = 16384
CONST_VAL = 0.1
VU = 1000000
VI = 100000

NC = 2
NS = 16
L = 16
NW = NC * NS

W = 512
NFULL_U = VU // W
NFULL_I = VI // W
ALIGN_I = 99840
PAD_U = 999936
PAD_I = 99968
CBUF = 2048
NCTX = B * OUT // NW // CBUF

_mesh = plsc.VectorSubcoreMesh(core_axis_name="c", subcore_axis_name="s")


@functools.partial(
    pl.kernel,
    mesh=_mesh,
    compiler_params=pltpu.CompilerParams(
        use_tc_tiling_on_sc=True, needs_layout_passes=False),
    out_type=(
        jax.ShapeDtypeStruct((B * OUT,), jnp.float32),
        jax.ShapeDtypeStruct((B * OUT,), jnp.float32),
        jax.ShapeDtypeStruct((B * OUT,), jnp.float32),
    ),
    scratch_types=[
        pltpu.VMEM((B,), jnp.int32),
        pltpu.VMEM((B + L,), jnp.int32),
        pltpu.VMEM((B + L,), jnp.int32),
        pltpu.VMEM((2, OUT, W), jnp.float32),
        pltpu.VMEM((L * OUT,), jnp.float32),
        pltpu.VMEM((L * OUT,), jnp.int32),
        pltpu.VMEM((CBUF,), jnp.float32),
        pltpu.SemaphoreType.DMA,
        pltpu.SemaphoreType.DMA,
        pltpu.SemaphoreType.DMA,
    ],
)
def _sc_encode(uid_hbm, iid_hbm, wut_hbm, wit_hbm, upad_hbm, ipad_hbm,
               ou_hbm, oi_hbm, oc_hbm,
               ids_v, mb_v, wb_v, buf3, ring_v, drain_v, cb_v,
               sem, sem_w, sem_c):
    wid = lax.axis_index("s") * NC + lax.axis_index("c")
    lanes = lax.iota(jnp.int32, L)

    cvec = jnp.full((L,), CONST_VAL, jnp.float32)

    def cfill(i, _):
        cb_v[pl.ds(i * L, L)] = cvec
        return ()

    lax.fori_loop(0, CBUF // L, cfill, (), unroll=8)
    base_c = wid * (B * OUT // NW)
    for j in range(NCTX):
        pltpu.async_copy(cb_v, oc_hbm.at[pl.ds(base_c + j * CBUF, CBUF)],
                         sem_c)

    def run_table(idx_hbm, wt_hbm, pad_hbm, out_hbm, nfull, align_base,
                  pad_base):
        pltpu.sync_copy(idx_hbm, ids_v)

        def pf(i, cnt):
            idv = ids_v[pl.ds(i * L, L)]
            bv = lanes + i * L
            m = (idv // W) % NW == wid
            pos = plsc.cumsum(m.astype(jnp.int32))
            dst = jnp.where(m, cnt + pos - 1, B)
            plsc.store_scatter(mb_v, [dst], bv)
            return cnt + jnp.max(pos)

        cnt = lax.fori_loop(0, B // L, pf, jnp.int32(0))
        nch = (cnt + L - 1) // L

        def process_window(par, base, match_w):
            parv = jnp.broadcast_to(par, (L,))

            def wf(j, k):
                bv = mb_v[pl.ds(j * L, L)]
                valid = (j * L + lanes) < cnt
                idv = plsc.load_gather(ids_v, [jnp.where(valid, bv, 0)])
                m = valid & (idv >= base) & (idv < base + match_w)
                pos = plsc.cumsum(m.astype(jnp.int32))
                dst = jnp.where(m, k + pos - 1, B)
                plsc.store_scatter(wb_v, [dst], bv)
                return k + jnp.max(pos)

            kw = lax.fori_loop(0, nch, wf, jnp.int32(0))

            def grp(t16, _):
                bv0 = wb_v[pl.ds(t16 * L, L)]
                rem = kw - t16 * L
                bvc = jnp.where(lanes < rem, bv0, 0)
                colv = jnp.where(lanes < rem,
                                 plsc.load_gather(ids_v, [bvc]) - base, 0)
                for l in range(L):
                    @pl.when(l < rem)
                    def _issue():
                        col = jnp.max(jnp.where(lanes == l, colv, 0))
                        b = jnp.max(jnp.where(lanes == l, bv0, 0))
                        for c0 in range(0, OUT, L):
                            ring_v[pl.ds(l * OUT + c0, L)] = plsc.load_gather(
                                buf3, [parv, lanes + c0,
                                       jnp.broadcast_to(col, (L,))])
                        pltpu.async_copy(ring_v.at[pl.ds(l * OUT, OUT)],
                                         out_hbm.at[pl.ds(b * OUT, OUT)], sem)

                    @pl.when(l >= rem)
                    def _dummy():
                        pltpu.async_copy(idx_hbm.at[pl.ds(l * OUT, OUT)],
                                         drain_v.at[pl.ds(l * OUT, OUT)], sem)
                pltpu.make_async_copy(idx_hbm.at[pl.ds(0, L * OUT)],
                                      drain_v, sem).wait()
                return ()

            lax.fori_loop(0, (kw + L - 1) // L, grp, ())

        @pl.when(wid == (pad_base // W) % NW)
        def _pad_station():
            pltpu.sync_copy(pad_hbm, buf3.at[0, :, pl.ds(0, 128)])
            process_window(jnp.int32(0), jnp.int32(pad_base),
                           W - (pad_base % W))

        if align_base is not None:
            @pl.when(wid == (align_base // W) % NW)
            def _align_station():
                pltpu.sync_copy(wt_hbm.at[:, pl.ds(ALIGN_I, 128)],
                                buf3.at[0, :, pl.ds(0, 128)])
                process_window(jnp.int32(0), jnp.int32(align_base), 128)

        nfull_me = (nfull + (NW - 1) - wid) // NW

        @pl.when(nfull_me > 0)
        def _prologue():
            pltpu.async_copy(wt_hbm.at[:, pl.ds(wid * W, W)], buf3.at[0],
                             sem_w)

        def wloop(lw, _):
            par = lw % 2
            g = wid + lw * NW
            pltpu.make_async_copy(wt_hbm.at[:, pl.ds(0, W)], buf3.at[par],
                                  sem_w).wait()

            @pl.when(lw + 1 < nfull_me)
            def _prefetch():
                pltpu.async_copy(wt_hbm.at[:, pl.ds((g + NW) * W, W)],
                                 buf3.at[1 - par], sem_w)

            process_window(par, g * W, W)
            return ()

        lax.fori_loop(0, nfull_me, wloop, ())

    run_table(uid_hbm, wut_hbm, upad_hbm, ou_hbm, NFULL_U, None, PAD_U)
    run_table(iid_hbm, wit_hbm, ipad_hbm, oi_hbm, NFULL_I, ALIGN_I, PAD_I)

    for j in range(NCTX):
        pltpu.make_async_copy(oc_hbm.at[pl.ds(0, CBUF)], cb_v, sem_c).wait()


@jax.jit
def kernel(user_ids, item_ids, context_ids, W_user, W_item):
    del context_ids
    upad = jnp.pad(W_user[PAD_U:].T, ((0, 0), (0, 128 - (VU - PAD_U))))
    ipad = jnp.pad(W_item[PAD_I:].T, ((0, 0), (0, 128 - (VI - PAD_I))))
    u_flat, i_flat, c_flat = _sc_encode(
        user_ids.astype(jnp.int32), item_ids.astype(jnp.int32),
        jnp.transpose(W_user), jnp.transpose(W_item), upad, ipad)
    user_emb = u_flat.reshape(B, OUT)
    item_emb = i_flat.reshape(B, OUT)
    ctx_emb = c_flat.reshape(OUT, B).T
    return user_emb, item_emb, ctx_emb
